# baseline (speedup 1.0000x reference)
import functools

import jax
import jax.numpy as jnp
from jax import lax
from jax.experimental import pallas as pl
from jax.experimental.pallas import tpu as pltpu
from jax.experimental.pallas import tpu_sc as plsc

NUM_EMBEDDINGS = 1000000
D = 64
B = 16384
L = 50

NW = 32
ROWS_PER_TILE = B // NW
C = 16
NCHUNK = ROWS_PER_TILE // C
LANES = 16
DV = D // LANES
NIDX = C * L


def _vrsqrt(x):
    i = plsc.bitcast(x, jnp.int32)
    i = jnp.int32(0x5F3759DF) - lax.shift_right_logical(i, 1)
    y = plsc.bitcast(i, jnp.float32)
    for _ in range(3):
        y = y * (1.5 - 0.5 * x * y * y)
    return y


_BCAST_DN = lax.GatherDimensionNumbers(
    offset_dims=(), collapsed_slice_dims=(0,), start_index_map=(0,))


def _bcast(vec, lane):
    idx = jnp.full((LANES, 1), lane, jnp.int32)
    return lax.gather(vec, idx, _BCAST_DN, (1,),
                      mode=lax.GatherScatterMode.PROMISE_IN_BOUNDS)


def _body(hashes_hbm, weights_hbm, table_hbm, out_hbm,
          idx_v, w_v, rows_v, out_v, gsem):
    wid = lax.axis_index("s") * 2 + lax.axis_index("c")
    tile_base = wid * ROWS_PER_TILE

    iota = lax.iota(jnp.int32, LANES)
    iota_d = iota * D

    def fetch(ci, p):
        row0 = tile_base + ci * C
        pltpu.sync_copy(hashes_hbm.at[pl.ds(row0, C), :], idx_v[p])
        pltpu.sync_copy(weights_hbm.at[pl.ds(row0 * L, NIDX)],
                        w_v[p].at[pl.ds(0, NIDX)])
        for j in range(C):
            pltpu.async_copy(table_hbm.at[idx_v[p].at[j]],
                             rows_v[p].at[pl.ds(j * L, L), :], gsem[p])

    def drain(p):
        for j in range(C):
            pltpu.make_async_copy(table_hbm.at[idx_v[p].at[j]],
                                  rows_v[p].at[pl.ds(j * L, L), :],
                                  gsem[p]).wait()

    def compute(ci, p):
        drain(p)

        lanes_d = [iota + d * LANES for d in range(DV)]

        def row_body(r, _):
            acc = [jnp.zeros((LANES,), jnp.float32) for _ in range(DV)]
            rl = jnp.full((LANES,), r * L, jnp.int32)
            obase = jnp.full((LANES,), r * D, jnp.int32) + iota
            for l in range(L):
                e = rl + l
                w = plsc.load_gather(w_v[p], [e])
                for d in range(DV):
                    v = plsc.load_gather(rows_v[p], [e, lanes_d[d]])
                    acc[d] = acc[d] + v * w
            for d in range(DV):
                plsc.store_scatter(out_v, [obase + d * LANES], acc[d])
            return ()

        lax.fori_loop(0, C, row_body, (), unroll=2)

        ss = jnp.zeros((LANES,), jnp.float32)
        for d in range(D):
            col = plsc.load_gather(out_v, [iota_d + d])
            ss = ss + col * col
        scale = _vrsqrt(jnp.maximum(ss, 1e-24))
        for d in range(D):
            idxs = iota_d + d
            col = plsc.load_gather(out_v, [idxs])
            plsc.store_scatter(out_v, [idxs], col * scale)

        row0 = tile_base + ci * C
        pltpu.sync_copy(out_v, out_hbm.at[pl.ds(row0 * D, C * D)])

    fetch(0, 0)

    def pair_body(k, _):
        c0 = k * 2
        fetch(c0 + 1, 1)
        compute(c0, 0)
        fetch(jnp.bitwise_and(c0 + 2, NCHUNK - 1), 0)
        compute(c0 + 1, 1)
        return ()

    lax.fori_loop(0, NCHUNK // 2, pair_body, (), unroll=False)
    drain(0)


@functools.partial(jax.jit, static_argnames=())
def _run(hashes, weights_flat, table):
    mesh = plsc.VectorSubcoreMesh(core_axis_name="c", subcore_axis_name="s")
    f = pl.kernel(
        _body,
        out_type=jax.ShapeDtypeStruct((B * D,), jnp.float32),
        mesh=mesh,
        scratch_types=[
            [pltpu.VMEM((C, L), jnp.int32) for _ in range(2)],
            [pltpu.VMEM((NIDX + LANES,), jnp.float32) for _ in range(2)],
            [pltpu.VMEM((NIDX, D), jnp.float32) for _ in range(2)],
            pltpu.VMEM((C * D,), jnp.float32),
            [pltpu.SemaphoreType.DMA for _ in range(2)],
        ],
        compiler_params=pltpu.CompilerParams(
            needs_layout_passes=False, use_tc_tiling_on_sc=False),
    )
    return f(hashes, weights_flat, table)


def kernel(feature_hashes, feature_weights, table):
    fh = feature_hashes.astype(jnp.int32)
    out_flat = _run(fh, feature_weights.reshape(B * L), table)
    return out_flat.reshape(B, D)


# speedup vs baseline: 1.0037x; 1.0037x over previous
import functools

import jax
import jax.numpy as jnp
from jax import lax
from jax.experimental import pallas as pl
from jax.experimental.pallas import tpu as pltpu
from jax.experimental.pallas import tpu_sc as plsc

NUM_EMBEDDINGS = 1000000
D = 64
B = 16384
L = 50

NW = 32
ROWS_PER_TILE = B // NW
C = 16
NCHUNK = ROWS_PER_TILE // C
LANES = 16
DV = D // LANES
NIDX = C * L


def _vrsqrt(x):
    i = plsc.bitcast(x, jnp.int32)
    i = jnp.int32(0x5F3759DF) - lax.shift_right_logical(i, 1)
    y = plsc.bitcast(i, jnp.float32)
    for _ in range(3):
        y = y * (1.5 - 0.5 * x * y * y)
    return y


_BCAST_DN = lax.GatherDimensionNumbers(
    offset_dims=(), collapsed_slice_dims=(0,), start_index_map=(0,))


def _bcast(vec, lane):
    idx = jnp.full((LANES, 1), lane, jnp.int32)
    return lax.gather(vec, idx, _BCAST_DN, (1,),
                      mode=lax.GatherScatterMode.PROMISE_IN_BOUNDS)


def _body(hashes_hbm, weights_hbm, table_hbm, out_hbm,
          idx_v, w_v, rows_v, out_v, gsem):
    wid = lax.axis_index("s") * 2 + lax.axis_index("c")
    tile_base = wid * ROWS_PER_TILE

    iota = lax.iota(jnp.int32, LANES)
    iota_d = iota * D

    def fetch(ci, p):
        row0 = tile_base + ci * C
        pltpu.sync_copy(hashes_hbm.at[pl.ds(row0, C), :], idx_v[p])
        pltpu.sync_copy(weights_hbm.at[pl.ds(row0 * L, NIDX)],
                        w_v[p].at[pl.ds(0, NIDX)])
        for j in range(C):
            pltpu.async_copy(table_hbm.at[idx_v[p].at[j]],
                             rows_v[p].at[pl.ds(j * L, L), :], gsem[p])

    def drain(p):
        for j in range(C):
            pltpu.make_async_copy(table_hbm.at[idx_v[p].at[j]],
                                  rows_v[p].at[pl.ds(j * L, L), :],
                                  gsem[p]).wait()

    def compute(ci, p):
        drain(p)

        lanes_d = [iota + d * LANES for d in range(DV)]

        def row_body(r, _):
            acc = [jnp.zeros((LANES,), jnp.float32) for _ in range(DV)]
            rl = jnp.full((LANES,), r * L, jnp.int32)
            obase = jnp.full((LANES,), r * D, jnp.int32) + iota
            for l in range(L):
                e = rl + l
                w = plsc.load_gather(w_v[p], [e])
                for d in range(DV):
                    v = plsc.load_gather(rows_v[p], [e, lanes_d[d]])
                    acc[d] = acc[d] + v * w
            for d in range(DV):
                plsc.store_scatter(out_v, [obase + d * LANES], acc[d])
            return ()

        lax.fori_loop(0, C, row_body, (), unroll=False)

        ss = jnp.zeros((LANES,), jnp.float32)
        for d in range(D):
            col = plsc.load_gather(out_v, [iota_d + d])
            ss = ss + col * col
        scale = _vrsqrt(jnp.maximum(ss, 1e-24))
        for d in range(D):
            idxs = iota_d + d
            col = plsc.load_gather(out_v, [idxs])
            plsc.store_scatter(out_v, [idxs], col * scale)

        row0 = tile_base + ci * C
        pltpu.sync_copy(out_v, out_hbm.at[pl.ds(row0 * D, C * D)])

    fetch(0, 0)

    def pair_body(k, _):
        c0 = k * 2
        fetch(c0 + 1, 1)
        compute(c0, 0)
        fetch(jnp.bitwise_and(c0 + 2, NCHUNK - 1), 0)
        compute(c0 + 1, 1)
        return ()

    lax.fori_loop(0, NCHUNK // 2, pair_body, (), unroll=False)
    drain(0)


@functools.partial(jax.jit, static_argnames=())
def _run(hashes, weights_flat, table):
    mesh = plsc.VectorSubcoreMesh(core_axis_name="c", subcore_axis_name="s")
    f = pl.kernel(
        _body,
        out_type=jax.ShapeDtypeStruct((B * D,), jnp.float32),
        mesh=mesh,
        scratch_types=[
            [pltpu.VMEM((C, L), jnp.int32) for _ in range(2)],
            [pltpu.VMEM((NIDX + LANES,), jnp.float32) for _ in range(2)],
            [pltpu.VMEM((NIDX, D), jnp.float32) for _ in range(2)],
            pltpu.VMEM((C * D,), jnp.float32),
            [pltpu.SemaphoreType.DMA for _ in range(2)],
        ],
        compiler_params=pltpu.CompilerParams(
            needs_layout_passes=False, use_tc_tiling_on_sc=False),
    )
    return f(hashes, weights_flat, table)


def kernel(feature_hashes, feature_weights, table):
    fh = feature_hashes.astype(jnp.int32)
    out_flat = _run(fh, feature_weights.reshape(B * L), table)
    return out_flat.reshape(B, D)
